# baseline (speedup 1.0000x reference)
import functools

import jax
import jax.numpy as jnp
from jax import lax
from jax.experimental import pallas as pl
from jax.experimental.pallas import tpu as pltpu
from jax.experimental.pallas import tpu_sc as plsc

_SIGMA = 9.0
_NC = 2
_NS = 16
_NW = _NC * _NS

_CHUNK = 8000
_GROUPS = _CHUNK // 16


def _sc_body(nchunk, trips, in_hbm, tgt_hbm, out_hbm, in_v, tgt_v, acc_s, acc_c, res_v):
    wid = lax.axis_index("s") * _NC + lax.axis_index("c")

    zeros = jnp.zeros((16,), jnp.float32)
    acc_s[...] = zeros
    acc_c[...] = zeros

    i16 = lax.iota(jnp.int32, 16)
    i3 = i16 * 3
    i2 = i16 * 2

    inv_sigma = jnp.float32(1.0 / _SIGMA)
    half_sigma = jnp.float32(0.5 * _SIGMA)
    half_inv_sigma = jnp.float32(0.5 / _SIGMA)
    one = jnp.float32(1.0)

    def group(j, carry):
        s, c = carry
        b3 = j * (3 * 16)
        b2 = j * (2 * 16)
        idx0 = b3 + i3
        cls = plsc.load_gather(tgt_v, [idx0])
        r1 = plsc.load_gather(tgt_v, [idx0 + 1])
        r2 = plsc.load_gather(tgt_v, [idx0 + 2])
        idxp = b2 + i2
        p1 = plsc.load_gather(in_v, [idxp])
        p2 = plsc.load_gather(in_v, [idxp + 1])
        d1 = jnp.abs(r1 - p1)
        d2 = jnp.abs(r2 - p2)
        l1 = jnp.where(d1 < inv_sigma, half_sigma * d1 * d1, d1 - half_inv_sigma)
        l2 = jnp.where(d2 < inv_sigma, half_sigma * d2 * d2, d2 - half_inv_sigma)
        m = cls == one
        s = s + jnp.where(m, l1 + l2, 0.0)
        c = c + jnp.where(m, one, 0.0)
        return s, c

    def chunk_step(i, _):
        g = wid + i * _NW

        @pl.when(g < nchunk)
        def _():
            pltpu.sync_copy(tgt_hbm.at[pl.ds(g * (3 * _CHUNK), 3 * _CHUNK)], tgt_v)
            pltpu.sync_copy(in_hbm.at[pl.ds(g * (2 * _CHUNK), 2 * _CHUNK)], in_v)
            s, c = lax.fori_loop(0, _GROUPS, group, (acc_s[...], acc_c[...]))
            acc_s[...] = s
            acc_c[...] = c

        return 0

    lax.fori_loop(0, trips, chunk_step, 0)

    res_v[pl.ds(0, 16)] = acc_s[...]
    res_v[pl.ds(16, 16)] = acc_c[...]
    pltpu.sync_copy(res_v, out_hbm.at[pl.ds(wid * 32, 32)])


def kernel(input_data, target):
    n = input_data.shape[1]
    assert n % _CHUNK == 0
    nchunk = n // _CHUNK
    trips = (nchunk + _NW - 1) // _NW

    in_flat = input_data.reshape(-1).astype(jnp.float32)
    tgt_flat = target.reshape(-1).astype(jnp.float32)

    mesh = plsc.VectorSubcoreMesh(core_axis_name="c", subcore_axis_name="s")
    partials = pl.kernel(
        functools.partial(_sc_body, nchunk, trips),
        out_type=jax.ShapeDtypeStruct((_NW * 32,), jnp.float32),
        mesh=mesh,
        compiler_params=pltpu.CompilerParams(needs_layout_passes=False),
        scratch_types=[
            pltpu.VMEM((2 * _CHUNK,), jnp.float32),
            pltpu.VMEM((3 * _CHUNK,), jnp.float32),
            pltpu.VMEM((16,), jnp.float32),
            pltpu.VMEM((16,), jnp.float32),
            pltpu.VMEM((32,), jnp.float32),
        ],
    )(in_flat, tgt_flat)

    p = partials.reshape(_NW, 2, 16)
    total = jnp.sum(p[:, 0, :])
    cnt = jnp.sum(p[:, 1, :])
    return jnp.where(cnt > 0, total / jnp.maximum(cnt, 1.0), jnp.float32(0.0))
